# baseline (speedup 1.0000x reference)
import functools

import jax
import jax.numpy as jnp
from jax import lax
from jax.experimental import pallas as pl
from jax.experimental.pallas import tpu as pltpu
from jax.experimental.pallas import tpu_sc as plsc

NC = 2
NS = 16
NW = NC * NS
L = 16

D = 128
V = 100000
B = 1024
G = 10
Z = 10
ROWS = D // NW

LAMB = 0.01


def _sc_body(rv_hbm, rd_hbm, wids_hbm, dids_hbm, nids_hbm,
             wpT_hbm, docsT_hbm, negsT_hbm,
             row_v, idx_v, did_v, out1a, out1b, outz_v,
             sem_w0, sem_w1, sem_d0, sem_d1, sem_n):
    wid = lax.axis_index("s") * NC + lax.axis_index("c")
    out1 = [out1a, out1b]
    sem_w = [sem_w0, sem_w1]
    sem_d = [sem_d0, sem_d1]

    pltpu.sync_copy(wids_hbm, idx_v)
    for r in range(ROWS):
        d = wid * ROWS + r
        pltpu.sync_copy(rv_hbm.at[d], row_v)
        ob = out1[r % 2]
        if r >= 2:
            pltpu.make_async_copy(ob, wpT_hbm.at[d - 2], sem_w[r % 2]).wait()

        def gbody(i, c):
            for u in range(2):
                base = (2 * i + u) * L
                vs = [plsc.load_gather(row_v, [idx_v[pl.ds(g * B + base, L)]])
                      for g in range(G)]
                while len(vs) > 1:
                    vs = [vs[j] + vs[j + 1] for j in range(0, len(vs) - 1, 2)] \
                        + ([vs[-1]] if len(vs) % 2 else [])
                ob[pl.ds(base, L)] = vs[0] * (1.0 / G)
            return c
        lax.fori_loop(0, B // L // 2, gbody, 0)
        pltpu.make_async_copy(ob, wpT_hbm.at[d], sem_w[r % 2]).start()

    for r in range(max(ROWS - 2, 0), ROWS):
        d = wid * ROWS + r
        pltpu.make_async_copy(out1[r % 2], wpT_hbm.at[d], sem_w[r % 2]).wait()

    pltpu.sync_copy(nids_hbm, idx_v)
    pltpu.sync_copy(dids_hbm, did_v)
    for r in range(ROWS):
        d = wid * ROWS + r
        pltpu.sync_copy(rd_hbm.at[d], row_v)
        ob = out1[r % 2]
        if r >= 2:
            pltpu.make_async_copy(ob, docsT_hbm.at[d - 2], sem_d[r % 2]).wait()
        if r >= 1:
            pltpu.make_async_copy(outz_v, negsT_hbm.at[d - 1], sem_n).wait()

        def dbody(i, c):
            for u in range(4):
                base = (4 * i + u) * L
                idx = did_v[pl.ds(base, L)]
                ob[pl.ds(base, L)] = plsc.load_gather(row_v, [idx])
            return c
        lax.fori_loop(0, B // L // 4, dbody, 0)
        pltpu.make_async_copy(ob, docsT_hbm.at[d], sem_d[r % 2]).start()

        def nbody(i, c):
            for z in range(Z):
                for u in range(2):
                    base = z * B + (2 * i + u) * L
                    idx = idx_v[pl.ds(base, L)]
                    outz_v[pl.ds(base, L)] = plsc.load_gather(row_v, [idx])
            return c
        lax.fori_loop(0, B // L // 2, nbody, 0)
        pltpu.make_async_copy(outz_v, negsT_hbm.at[d], sem_n).start()

    for r in range(max(ROWS - 2, 0), ROWS):
        d = wid * ROWS + r
        pltpu.make_async_copy(out1[r % 2], docsT_hbm.at[d], sem_d[r % 2]).wait()
    pltpu.make_async_copy(outz_v, negsT_hbm.at[wid * ROWS + ROWS - 1],
                          sem_n).wait()


@functools.cache
def _get_sc_call():
  return pl.kernel(
    _sc_body,
    out_type=(
        jax.ShapeDtypeStruct((D, B), jnp.float32),
        jax.ShapeDtypeStruct((D, B), jnp.float32),
        jax.ShapeDtypeStruct((D, Z * B), jnp.float32),
    ),
    mesh=plsc.VectorSubcoreMesh(
        core_axis_name="c", subcore_axis_name="s",
        num_cores=NC, num_subcores=NS),
    compiler_params=pltpu.CompilerParams(needs_layout_passes=False),
    scratch_types=[
        pltpu.VMEM((V,), jnp.float32),
        pltpu.VMEM((G * B,), jnp.int32),
        pltpu.VMEM((B,), jnp.int32),
        pltpu.VMEM((B,), jnp.float32),
        pltpu.VMEM((B,), jnp.float32),
        pltpu.VMEM((Z * B,), jnp.float32),
        pltpu.SemaphoreType.DMA,
        pltpu.SemaphoreType.DMA,
        pltpu.SemaphoreType.DMA,
        pltpu.SemaphoreType.DMA,
        pltpu.SemaphoreType.DMA,
    ],
  )


SSQ_RB = 32
SSQ_NB = D // SSQ_RB


def _ssq_body(rd_ref, out_ref, acc_ref):
    i = pl.program_id(0)

    @pl.when(i == 0)
    def _init():
        acc_ref[0, 0] = 0.0

    x = rd_ref[...]
    acc_ref[0, 0] += jnp.sum(x * x)

    @pl.when(i == SSQ_NB - 1)
    def _fin():
        out_ref[...] = jnp.broadcast_to(acc_ref[0, 0], (1, 1))


def _tc_ssq(rd, interpret=False):
    return pl.pallas_call(
        _ssq_body,
        grid=(SSQ_NB,),
        in_specs=[pl.BlockSpec((SSQ_RB, V), lambda i: (i, 0))],
        out_specs=pl.BlockSpec((1, 1), lambda i: (0, 0)),
        out_shape=jax.ShapeDtypeStruct((1, 1), jnp.float32),
        scratch_shapes=[pltpu.SMEM((1, 1), jnp.float32)],
        interpret=interpret,
    )(rd)


def _tc_body(wpT_ref, docsT_ref, negsT_ref, ssq_ref, proj_ref, beta_ref,
             out_ref):
    wpT = wpT_ref[...]
    n2 = jnp.sum(wpT * wpT, axis=0, keepdims=True)
    normedT = wpT / jnp.sqrt(n2)
    tT = jnp.dot(proj_ref[...], normedT,
                 preferred_element_type=jnp.float32)
    mean = jnp.mean(tT, axis=1, keepdims=True)
    var = jnp.sum((tT - mean) ** 2, axis=1, keepdims=True) / (B - 1)
    std = jnp.sqrt(var)
    t = jnp.clip((tT - mean) / jnp.sqrt(std) + beta_ref[...], -1.0, 1.0)

    pos = jnp.sum(t * docsT_ref[...], axis=0, keepdims=True)
    p_pos = jnp.minimum(jax.nn.sigmoid(pos), 0.999)
    acc = Z * jnp.log(p_pos)
    for z in range(Z):
        dz = jnp.sum(t * negsT_ref[:, z * B:(z + 1) * B], axis=0,
                     keepdims=True)
        p = jnp.minimum(jax.nn.sigmoid(dz), 0.999)
        acc = acc + jnp.log(jnp.maximum(1.0 - p, 0.01))

    total = jnp.sum(acc) * ((Z + 1) / (2 * Z))
    reg = ssq_ref[0, 0] + jnp.sum(proj_ref[...] * proj_ref[...])
    loss = total / B + LAMB / (2 * B) * reg
    out_ref[...] = jnp.broadcast_to(loss, (1, 1))


def kernel(rv, rd, proj, beta, word_ids, doc_ids, nsample_ids):
    widsT = jnp.transpose(word_ids).astype(jnp.int32).reshape(-1)
    nidsT = jnp.transpose(nsample_ids).astype(jnp.int32).reshape(-1)
    dids = doc_ids.astype(jnp.int32)

    ssq = _tc_ssq(rd)

    wpT, docsT, negsT = _get_sc_call()(rv, rd, widsT, dids, nidsT)

    out = pl.pallas_call(
        _tc_body,
        out_shape=jax.ShapeDtypeStruct((1, 1), jnp.float32),
    )(wpT, docsT, negsT, ssq, proj, beta)
    return out[0, 0]


# speedup vs baseline: 1.0017x; 1.0017x over previous
import functools

import jax
import jax.numpy as jnp
from jax import lax
from jax.experimental import pallas as pl
from jax.experimental.pallas import tpu as pltpu
from jax.experimental.pallas import tpu_sc as plsc

NC = 2
NS = 16
NW = NC * NS
L = 16

D = 128
V = 100000
B = 1024
G = 10
Z = 10
ROWS = D // NW

LAMB = 0.01


def _sc_body(rv_hbm, rd_hbm, wids_hbm, dids_hbm, nids_hbm,
             wpT_hbm, docsT_hbm, negsT_hbm,
             row_v, idx_v, did_v, out1a, out1b, outz_v,
             sem_w0, sem_w1, sem_d0, sem_d1, sem_n):
    wid = lax.axis_index("s") * NC + lax.axis_index("c")
    out1 = [out1a, out1b]
    sem_w = [sem_w0, sem_w1]
    sem_d = [sem_d0, sem_d1]

    pltpu.sync_copy(wids_hbm, idx_v)
    for r in range(ROWS):
        d = wid * ROWS + r
        pltpu.sync_copy(rv_hbm.at[d], row_v)
        ob = out1[r % 2]
        if r >= 2:
            pltpu.make_async_copy(ob, wpT_hbm.at[d - 2], sem_w[r % 2]).wait()

        def gbody(i, c):
            for u in range(2):
                base = (2 * i + u) * L
                vs = [plsc.load_gather(row_v, [idx_v[pl.ds(g * B + base, L)]])
                      for g in range(G)]
                while len(vs) > 1:
                    vs = [vs[j] + vs[j + 1] for j in range(0, len(vs) - 1, 2)] \
                        + ([vs[-1]] if len(vs) % 2 else [])
                ob[pl.ds(base, L)] = vs[0] * (1.0 / G)
            return c
        lax.fori_loop(0, B // L // 2, gbody, 0)
        pltpu.make_async_copy(ob, wpT_hbm.at[d], sem_w[r % 2]).start()

    for r in range(max(ROWS - 2, 0), ROWS):
        d = wid * ROWS + r
        pltpu.make_async_copy(out1[r % 2], wpT_hbm.at[d], sem_w[r % 2]).wait()

    pltpu.sync_copy(nids_hbm, idx_v)
    pltpu.sync_copy(dids_hbm, did_v)
    for r in range(ROWS):
        d = wid * ROWS + r
        pltpu.sync_copy(rd_hbm.at[d], row_v)
        ob = out1[r % 2]
        if r >= 2:
            pltpu.make_async_copy(ob, docsT_hbm.at[d - 2], sem_d[r % 2]).wait()
        if r >= 1:
            pltpu.make_async_copy(outz_v, negsT_hbm.at[d - 1], sem_n).wait()

        def dbody(i, c):
            for u in range(4):
                base = (4 * i + u) * L
                idx = did_v[pl.ds(base, L)]
                ob[pl.ds(base, L)] = plsc.load_gather(row_v, [idx])
            return c
        lax.fori_loop(0, B // L // 4, dbody, 0)
        pltpu.make_async_copy(ob, docsT_hbm.at[d], sem_d[r % 2]).start()

        def nbody(i, c):
            for z in range(Z):
                for u in range(2):
                    base = z * B + (2 * i + u) * L
                    idx = idx_v[pl.ds(base, L)]
                    outz_v[pl.ds(base, L)] = plsc.load_gather(row_v, [idx])
            return c
        lax.fori_loop(0, B // L // 2, nbody, 0)
        pltpu.make_async_copy(outz_v, negsT_hbm.at[d], sem_n).start()

    for r in range(max(ROWS - 2, 0), ROWS):
        d = wid * ROWS + r
        pltpu.make_async_copy(out1[r % 2], docsT_hbm.at[d], sem_d[r % 2]).wait()
    pltpu.make_async_copy(outz_v, negsT_hbm.at[wid * ROWS + ROWS - 1],
                          sem_n).wait()


@functools.cache
def _get_sc_call():
  return pl.kernel(
    _sc_body,
    out_type=(
        jax.ShapeDtypeStruct((D, B), jnp.float32),
        jax.ShapeDtypeStruct((D, B), jnp.float32),
        jax.ShapeDtypeStruct((D, Z * B), jnp.float32),
    ),
    mesh=plsc.VectorSubcoreMesh(
        core_axis_name="c", subcore_axis_name="s",
        num_cores=NC, num_subcores=NS),
    compiler_params=pltpu.CompilerParams(needs_layout_passes=False, use_tc_tiling_on_sc=True),
    scratch_types=[
        pltpu.VMEM((V,), jnp.float32),
        pltpu.VMEM((G * B,), jnp.int32),
        pltpu.VMEM((B,), jnp.int32),
        pltpu.VMEM((B,), jnp.float32),
        pltpu.VMEM((B,), jnp.float32),
        pltpu.VMEM((Z * B,), jnp.float32),
        pltpu.SemaphoreType.DMA,
        pltpu.SemaphoreType.DMA,
        pltpu.SemaphoreType.DMA,
        pltpu.SemaphoreType.DMA,
        pltpu.SemaphoreType.DMA,
    ],
  )


SSQ_RB = 32
SSQ_NB = D // SSQ_RB


def _ssq_body(rd_ref, out_ref, acc_ref):
    i = pl.program_id(0)

    @pl.when(i == 0)
    def _init():
        acc_ref[0, 0] = 0.0

    x = rd_ref[...]
    acc_ref[0, 0] += jnp.sum(x * x)

    @pl.when(i == SSQ_NB - 1)
    def _fin():
        out_ref[...] = jnp.broadcast_to(acc_ref[0, 0], (1, 1))


def _tc_ssq(rd, interpret=False):
    return pl.pallas_call(
        _ssq_body,
        grid=(SSQ_NB,),
        in_specs=[pl.BlockSpec((SSQ_RB, V), lambda i: (i, 0))],
        out_specs=pl.BlockSpec((1, 1), lambda i: (0, 0)),
        out_shape=jax.ShapeDtypeStruct((1, 1), jnp.float32),
        scratch_shapes=[pltpu.SMEM((1, 1), jnp.float32)],
        interpret=interpret,
    )(rd)


def _tc_body(wpT_ref, docsT_ref, negsT_ref, ssq_ref, proj_ref, beta_ref,
             out_ref):
    wpT = wpT_ref[...]
    n2 = jnp.sum(wpT * wpT, axis=0, keepdims=True)
    normedT = wpT / jnp.sqrt(n2)
    tT = jnp.dot(proj_ref[...], normedT,
                 preferred_element_type=jnp.float32)
    mean = jnp.mean(tT, axis=1, keepdims=True)
    var = jnp.sum((tT - mean) ** 2, axis=1, keepdims=True) / (B - 1)
    std = jnp.sqrt(var)
    t = jnp.clip((tT - mean) / jnp.sqrt(std) + beta_ref[...], -1.0, 1.0)

    pos = jnp.sum(t * docsT_ref[...], axis=0, keepdims=True)
    p_pos = jnp.minimum(jax.nn.sigmoid(pos), 0.999)
    acc = Z * jnp.log(p_pos)
    for z in range(Z):
        dz = jnp.sum(t * negsT_ref[:, z * B:(z + 1) * B], axis=0,
                     keepdims=True)
        p = jnp.minimum(jax.nn.sigmoid(dz), 0.999)
        acc = acc + jnp.log(jnp.maximum(1.0 - p, 0.01))

    total = jnp.sum(acc) * ((Z + 1) / (2 * Z))
    reg = ssq_ref[0, 0] + jnp.sum(proj_ref[...] * proj_ref[...])
    loss = total / B + LAMB / (2 * B) * reg
    out_ref[...] = jnp.broadcast_to(loss, (1, 1))


def kernel(rv, rd, proj, beta, word_ids, doc_ids, nsample_ids):
    widsT = jnp.transpose(word_ids).astype(jnp.int32).reshape(-1)
    nidsT = jnp.transpose(nsample_ids).astype(jnp.int32).reshape(-1)
    dids = doc_ids.astype(jnp.int32)

    ssq = _tc_ssq(rd)

    wpT, docsT, negsT = _get_sc_call()(rv, rd, widsT, dids, nidsT)

    out = pl.pallas_call(
        _tc_body,
        out_shape=jax.ShapeDtypeStruct((1, 1), jnp.float32),
    )(wpT, docsT, negsT, ssq, proj, beta)
    return out[0, 0]


# speedup vs baseline: 1.0846x; 1.0828x over previous
import functools

import jax
import jax.numpy as jnp
from jax import lax
from jax.experimental import pallas as pl
from jax.experimental.pallas import tpu as pltpu
from jax.experimental.pallas import tpu_sc as plsc

NC = 2
NS = 16
NW = NC * NS
L = 16

D = 128
V = 100000
B = 1024
G = 10
Z = 10
ROWS = D // NW

LAMB = 0.01


def _sc_rv_body(rv_hbm, wids_hbm, wpT_hbm,
                row_v, idx_v, out1a, out1b, sem_w0, sem_w1):
    wid = lax.axis_index("s") * NC + lax.axis_index("c")
    out1 = [out1a, out1b]
    sem_w = [sem_w0, sem_w1]

    pltpu.sync_copy(wids_hbm, idx_v)
    for r in range(ROWS):
        d = wid * ROWS + r
        pltpu.sync_copy(rv_hbm.at[d], row_v)
        ob = out1[r % 2]
        if r >= 2:
            pltpu.make_async_copy(ob, wpT_hbm.at[d - 2], sem_w[r % 2]).wait()

        def gbody(i, c):
            for u in range(2):
                base = (2 * i + u) * L
                vs = [plsc.load_gather(row_v, [idx_v[pl.ds(g * B + base, L)]])
                      for g in range(G)]
                while len(vs) > 1:
                    vs = [vs[j] + vs[j + 1] for j in range(0, len(vs) - 1, 2)] \
                        + ([vs[-1]] if len(vs) % 2 else [])
                ob[pl.ds(base, L)] = vs[0] * (1.0 / G)
            return c
        lax.fori_loop(0, B // L // 2, gbody, 0)
        pltpu.make_async_copy(ob, wpT_hbm.at[d], sem_w[r % 2]).start()

    for r in range(max(ROWS - 2, 0), ROWS):
        d = wid * ROWS + r
        pltpu.make_async_copy(out1[r % 2], wpT_hbm.at[d], sem_w[r % 2]).wait()


def _sc_rd_body(rd_hbm, dids_hbm, nids_hbm, docsT_hbm, negsT_hbm,
                row_v, idx_v, did_v, out1a, out1b, outz_v,
                sem_d0, sem_d1, sem_n):
    wid = lax.axis_index("s") * NC + lax.axis_index("c")
    out1 = [out1a, out1b]
    sem_d = [sem_d0, sem_d1]

    pltpu.sync_copy(nids_hbm, idx_v)
    pltpu.sync_copy(dids_hbm, did_v)
    for r in range(ROWS):
        d = wid * ROWS + r
        pltpu.sync_copy(rd_hbm.at[d], row_v)
        ob = out1[r % 2]
        if r >= 2:
            pltpu.make_async_copy(ob, docsT_hbm.at[d - 2], sem_d[r % 2]).wait()
        if r >= 1:
            pltpu.make_async_copy(outz_v, negsT_hbm.at[d - 1], sem_n).wait()

        def dbody(i, c):
            for u in range(4):
                base = (4 * i + u) * L
                idx = did_v[pl.ds(base, L)]
                ob[pl.ds(base, L)] = plsc.load_gather(row_v, [idx])
            return c
        lax.fori_loop(0, B // L // 4, dbody, 0)
        pltpu.make_async_copy(ob, docsT_hbm.at[d], sem_d[r % 2]).start()

        def nbody(i, c):
            for z in range(Z):
                for u in range(2):
                    base = z * B + (2 * i + u) * L
                    idx = idx_v[pl.ds(base, L)]
                    outz_v[pl.ds(base, L)] = plsc.load_gather(row_v, [idx])
            return c
        lax.fori_loop(0, B // L // 2, nbody, 0)
        pltpu.make_async_copy(outz_v, negsT_hbm.at[d], sem_n).start()

    for r in range(max(ROWS - 2, 0), ROWS):
        d = wid * ROWS + r
        pltpu.make_async_copy(out1[r % 2], docsT_hbm.at[d], sem_d[r % 2]).wait()
    pltpu.make_async_copy(outz_v, negsT_hbm.at[wid * ROWS + ROWS - 1],
                          sem_n).wait()


@functools.cache
def _get_sc_calls():
  mesh = plsc.VectorSubcoreMesh(
      core_axis_name="c", subcore_axis_name="s",
      num_cores=NC, num_subcores=NS)
  cparams = pltpu.CompilerParams(needs_layout_passes=False)
  rv_call = pl.kernel(
      _sc_rv_body,
      out_type=jax.ShapeDtypeStruct((D, B), jnp.float32),
      mesh=mesh,
      compiler_params=cparams,
      scratch_types=[
          pltpu.VMEM((V,), jnp.float32),
          pltpu.VMEM((G * B,), jnp.int32),
          pltpu.VMEM((B,), jnp.float32),
          pltpu.VMEM((B,), jnp.float32),
          pltpu.SemaphoreType.DMA,
          pltpu.SemaphoreType.DMA,
      ],
  )
  rd_call = pl.kernel(
      _sc_rd_body,
      out_type=(
          jax.ShapeDtypeStruct((D, B), jnp.float32),
          jax.ShapeDtypeStruct((D, Z * B), jnp.float32),
      ),
      mesh=mesh,
      compiler_params=cparams,
      scratch_types=[
          pltpu.VMEM((V,), jnp.float32),
          pltpu.VMEM((Z * B,), jnp.int32),
          pltpu.VMEM((B,), jnp.int32),
          pltpu.VMEM((B,), jnp.float32),
          pltpu.VMEM((B,), jnp.float32),
          pltpu.VMEM((Z * B,), jnp.float32),
          pltpu.SemaphoreType.DMA,
          pltpu.SemaphoreType.DMA,
          pltpu.SemaphoreType.DMA,
      ],
  )
  return rv_call, rd_call


SSQ_RB = 32
SSQ_NB = D // SSQ_RB


def _ssq_body(rd_ref, out_ref, acc_ref):
    i = pl.program_id(0)

    @pl.when(i == 0)
    def _init():
        acc_ref[0, 0] = 0.0

    x = rd_ref[...]
    acc_ref[0, 0] += jnp.sum(x * x)

    @pl.when(i == SSQ_NB - 1)
    def _fin():
        out_ref[...] = jnp.broadcast_to(acc_ref[0, 0], (1, 1))


def _tc_ssq(rd, interpret=False):
    return pl.pallas_call(
        _ssq_body,
        grid=(SSQ_NB,),
        in_specs=[pl.BlockSpec((SSQ_RB, V), lambda i: (i, 0))],
        out_specs=pl.BlockSpec((1, 1), lambda i: (0, 0)),
        out_shape=jax.ShapeDtypeStruct((1, 1), jnp.float32),
        scratch_shapes=[pltpu.SMEM((1, 1), jnp.float32)],
        interpret=interpret,
    )(rd)


def _tc_body(wpT_ref, docsT_ref, negsT_ref, ssq_ref, proj_ref, beta_ref,
             out_ref):
    wpT = wpT_ref[...]
    n2 = jnp.sum(wpT * wpT, axis=0, keepdims=True)
    normedT = wpT / jnp.sqrt(n2)
    tT = jnp.dot(proj_ref[...], normedT,
                 preferred_element_type=jnp.float32)
    mean = jnp.mean(tT, axis=1, keepdims=True)
    var = jnp.sum((tT - mean) ** 2, axis=1, keepdims=True) / (B - 1)
    std = jnp.sqrt(var)
    t = jnp.clip((tT - mean) / jnp.sqrt(std) + beta_ref[...], -1.0, 1.0)

    pos = jnp.sum(t * docsT_ref[...], axis=0, keepdims=True)
    p_pos = jnp.minimum(jax.nn.sigmoid(pos), 0.999)
    acc = Z * jnp.log(p_pos)
    for z in range(Z):
        dz = jnp.sum(t * negsT_ref[:, z * B:(z + 1) * B], axis=0,
                     keepdims=True)
        p = jnp.minimum(jax.nn.sigmoid(dz), 0.999)
        acc = acc + jnp.log(jnp.maximum(1.0 - p, 0.01))

    total = jnp.sum(acc) * ((Z + 1) / (2 * Z))
    reg = ssq_ref[0, 0] + jnp.sum(proj_ref[...] * proj_ref[...])
    loss = total / B + LAMB / (2 * B) * reg
    out_ref[...] = jnp.broadcast_to(loss, (1, 1))


def kernel(rv, rd, proj, beta, word_ids, doc_ids, nsample_ids):
    widsT = jnp.transpose(word_ids).astype(jnp.int32).reshape(-1)
    nidsT = jnp.transpose(nsample_ids).astype(jnp.int32).reshape(-1)
    dids = doc_ids.astype(jnp.int32)

    rv_call, rd_call = _get_sc_calls()
    wpT = rv_call(rv, widsT)
    docsT, negsT = rd_call(rd, dids, nidsT)
    ssq = _tc_ssq(rd)

    out = pl.pallas_call(
        _tc_body,
        out_shape=jax.ShapeDtypeStruct((1, 1), jnp.float32),
    )(wpT, docsT, negsT, ssq, proj, beta)
    return out[0, 0]
